# baseline (speedup 1.0000x reference)
import functools

import jax
import jax.numpy as jnp
from jax import lax
from jax.experimental import pallas as pl
from jax.experimental.pallas import tpu as pltpu
from jax.experimental.pallas import tpu_sc as plsc

_N_NODES = 100000
_E = 1600000
_NC = 2
_NS = 16
_NW = _NC * _NS
_CHUNK = 256
_EPW = _E // _NW
_NCHUNKS = -(-_EPW // _CHUNK)
_NCHUNKS += _NCHUNKS % 2


def _rsqrt(x):
    i = plsc.bitcast(x, jnp.int32)
    i = jnp.int32(0x5F3759DF) - (i >> 1)
    y = plsc.bitcast(i, jnp.float32)
    xh = x * 0.5
    for _ in range(3):
        y = y * (1.5 - xh * y * y)
    return y


def _cvec(v):
    return jnp.full((16,), v, jnp.int32)


def _compute_chunk(xi_b, xj_b, ob):
    def block(b, carry):
        rows = lax.iota(jnp.int32, 16) + b * 16
        qi = [[plsc.load_gather(xi_b, [rows, _cvec(4 * p + c)])
               for c in range(4)] for p in range(4)]
        qj = [[plsc.load_gather(xj_b, [rows, _cvec(4 * p + c)])
               for c in range(4)] for p in range(4)]
        for p in range(4):
            aw, ax, ay, az = qi[p]
            bw, bx, by, bz = qj[p]
            w = aw * bw + ax * bx + ay * by + az * bz
            x = ax * bw - aw * bx - ay * bz + az * by
            y = ay * bw - aw * by + ax * bz - az * bx
            z = az * bw - aw * bz - ax * by + ay * bx
            r = _rsqrt(w * w + x * x + y * y + z * z)
            for c, v in enumerate((w * r, x * r, y * r, z * r)):
                plsc.store_scatter(ob, [rows, _cvec(p), _cvec(c)], v)
        return carry

    lax.fori_loop(0, _CHUNK // 16, block, 0, unroll=4)


def _sc_body(table_hbm, edges_hbm, out_hbm,
             ii0, ij0, ii1, ij1, xi0, xj0, xi1, xj1, ob,
             sg0, sg1):
    wid = lax.axis_index("s") * _NC + lax.axis_index("c")
    wbase = wid * _EPW
    n = _NCHUNKS

    def idx_load(slot_ii, slot_ij, base):
        pltpu.sync_copy(edges_hbm.at[0, pl.ds(base, _CHUNK)], slot_ii)
        pltpu.sync_copy(edges_hbm.at[1, pl.ds(base, _CHUNK)], slot_ij)

    def gather_issue(slot_ii, slot_ij, xi_b, xj_b, sem):
        pltpu.async_copy(table_hbm.at[slot_ii], xi_b, sem)
        pltpu.async_copy(table_hbm.at[slot_ij], xj_b, sem)

    def gather_wait(slot_ii, slot_ij, xi_b, xj_b, sem):
        pltpu.make_async_copy(table_hbm.at[slot_ii], xi_b, sem).wait()
        pltpu.make_async_copy(table_hbm.at[slot_ij], xj_b, sem).wait()

    idx_load(ii0, ij0, wbase)
    gather_issue(ii0, ij0, xi0, xj0, sg0)

    def half_iter(g, cur, nxt):
        (ii_c, ij_c, xi_c, xj_c, sg_c) = cur
        (ii_n, ij_n, xi_n, xj_n, sg_n) = nxt
        base_g = wbase + jnp.minimum(g * _CHUNK, _EPW - _CHUNK)
        base_n = wbase + jnp.minimum((g + 1) * _CHUNK, _EPW - _CHUNK)
        idx_load(ii_n, ij_n, base_n)
        gather_issue(ii_n, ij_n, xi_n, xj_n, sg_n)
        gather_wait(ii_c, ij_c, xi_c, xj_c, sg_c)
        _compute_chunk(xi_c, xj_c, ob)
        pltpu.sync_copy(ob, out_hbm.at[pl.ds(base_g, _CHUNK)])

    slot0 = (ii0, ij0, xi0, xj0, sg0)
    slot1 = (ii1, ij1, xi1, xj1, sg1)

    def loop_body(t, carry):
        half_iter(2 * t, slot0, slot1)
        half_iter(2 * t + 1, slot1, slot0)
        return carry

    lax.fori_loop(0, n // 2, loop_body, 0)

    gather_wait(ii0, ij0, xi0, xj0, sg0)


def kernel(particles, edges):
    table = particles.reshape(_N_NODES, 16)
    mesh = plsc.VectorSubcoreMesh(core_axis_name="c", subcore_axis_name="s")
    run = functools.partial(
        pl.kernel,
        mesh=mesh,
        compiler_params=pltpu.CompilerParams(
            use_tc_tiling_on_sc=False, needs_layout_passes=False),
        out_type=jax.ShapeDtypeStruct((_E, 4, 4), jnp.float32),
        scratch_types=[
            pltpu.VMEM((_CHUNK,), jnp.int32),
            pltpu.VMEM((_CHUNK,), jnp.int32),
            pltpu.VMEM((_CHUNK,), jnp.int32),
            pltpu.VMEM((_CHUNK,), jnp.int32),
            pltpu.VMEM((_CHUNK, 16), jnp.float32),
            pltpu.VMEM((_CHUNK, 16), jnp.float32),
            pltpu.VMEM((_CHUNK, 16), jnp.float32),
            pltpu.VMEM((_CHUNK, 16), jnp.float32),
            pltpu.VMEM((_CHUNK, 4, 4), jnp.float32),
            pltpu.SemaphoreType.DMA,
            pltpu.SemaphoreType.DMA,
        ],
    )(_sc_body)
    return run(table, edges)


# speedup vs baseline: 3.8360x; 3.8360x over previous
import functools

import jax
import jax.numpy as jnp
from jax import lax
from jax.experimental import pallas as pl
from jax.experimental.pallas import tpu as pltpu
from jax.experimental.pallas import tpu_sc as plsc

_N_NODES = 100000
_E = 1600000
_NC = 2
_NS = 16
_NW = _NC * _NS
_CHUNK = 256
_EPW = _E // _NW
_NCHUNKS = -(-_EPW // _CHUNK)
_NCHUNKS += _NCHUNKS % 2


def _rsqrt(x):
    i = plsc.bitcast(x, jnp.int32)
    i = jnp.int32(0x5F3759DF) - (i >> 1)
    y = plsc.bitcast(i, jnp.float32)
    xh = x * 0.5
    for _ in range(3):
        y = y * (1.5 - xh * y * y)
    return y


def _cvec(v):
    return jnp.full((16,), v, jnp.int32)


def _compute_chunk(xi_b, xj_b, ob):
    def block(b, carry):
        rows = lax.iota(jnp.int32, 16) + b * 16
        qi = [[plsc.load_gather(xi_b, [rows, _cvec(4 * p + c)])
               for c in range(4)] for p in range(4)]
        qj = [[plsc.load_gather(xj_b, [rows, _cvec(4 * p + c)])
               for c in range(4)] for p in range(4)]
        for p in range(4):
            aw, ax, ay, az = qi[p]
            bw, bx, by, bz = qj[p]
            w = aw * bw + ax * bx + ay * by + az * bz
            x = ax * bw - aw * bx - ay * bz + az * by
            y = ay * bw - aw * by + ax * bz - az * bx
            z = az * bw - aw * bz - ax * by + ay * bx
            r = _rsqrt(w * w + x * x + y * y + z * z)
            for c, v in enumerate((w * r, x * r, y * r, z * r)):
                plsc.store_scatter(ob, [rows, _cvec(4 * p + c)], v)
        return carry

    lax.fori_loop(0, _CHUNK // 16, block, 0, unroll=4)


_ROWS_PER_SUB = _N_NODES // _NS


def _sc_body(table_hbm, edges_hbm, out_hbm,
             table_sh, ii0, ij0, ii1, ij1, xi0, xj0, xi1, xj1, ob,
             sg0, sg1):
    sid = lax.axis_index("s")
    wid = sid * _NC + lax.axis_index("c")
    wbase = wid * _EPW
    n = _NCHUNKS

    srow = sid * _ROWS_PER_SUB
    pltpu.sync_copy(table_hbm.at[pl.ds(srow, _ROWS_PER_SUB)],
                    table_sh.at[pl.ds(srow, _ROWS_PER_SUB)])
    plsc.subcore_barrier()

    def idx_load(slot_ii, slot_ij, base):
        pltpu.sync_copy(edges_hbm.at[0, pl.ds(base, _CHUNK)], slot_ii)
        pltpu.sync_copy(edges_hbm.at[1, pl.ds(base, _CHUNK)], slot_ij)

    def gather_issue(slot_ii, slot_ij, xi_b, xj_b, sem):
        pltpu.async_copy(table_sh.at[slot_ii], xi_b, sem)
        pltpu.async_copy(table_sh.at[slot_ij], xj_b, sem)

    def gather_wait(slot_ii, slot_ij, xi_b, xj_b, sem):
        pltpu.make_async_copy(table_sh.at[slot_ii], xi_b, sem).wait()
        pltpu.make_async_copy(table_sh.at[slot_ij], xj_b, sem).wait()

    idx_load(ii0, ij0, wbase)
    gather_issue(ii0, ij0, xi0, xj0, sg0)

    def half_iter(g, cur, nxt):
        (ii_c, ij_c, xi_c, xj_c, sg_c) = cur
        (ii_n, ij_n, xi_n, xj_n, sg_n) = nxt
        base_g = wbase + jnp.minimum(g * _CHUNK, _EPW - _CHUNK)
        base_n = wbase + jnp.minimum((g + 1) * _CHUNK, _EPW - _CHUNK)
        idx_load(ii_n, ij_n, base_n)
        gather_issue(ii_n, ij_n, xi_n, xj_n, sg_n)
        gather_wait(ii_c, ij_c, xi_c, xj_c, sg_c)
        _compute_chunk(xi_c, xj_c, ob)
        pltpu.sync_copy(ob, out_hbm.at[pl.ds(base_g, _CHUNK)])

    slot0 = (ii0, ij0, xi0, xj0, sg0)
    slot1 = (ii1, ij1, xi1, xj1, sg1)

    def loop_body(t, carry):
        half_iter(2 * t, slot0, slot1)
        half_iter(2 * t + 1, slot1, slot0)
        return carry

    lax.fori_loop(0, n // 2, loop_body, 0)

    gather_wait(ii0, ij0, xi0, xj0, sg0)


def kernel(particles, edges):
    table = particles.reshape(_N_NODES, 16)
    mesh = plsc.VectorSubcoreMesh(core_axis_name="c", subcore_axis_name="s")
    run = functools.partial(
        pl.kernel,
        mesh=mesh,
        compiler_params=pltpu.CompilerParams(
            use_tc_tiling_on_sc=False, needs_layout_passes=False),
        out_type=jax.ShapeDtypeStruct((_E, 16), jnp.float32),
        scratch_types=[
            pltpu.VMEM_SHARED((_N_NODES, 16), jnp.float32),
            pltpu.VMEM((_CHUNK,), jnp.int32),
            pltpu.VMEM((_CHUNK,), jnp.int32),
            pltpu.VMEM((_CHUNK,), jnp.int32),
            pltpu.VMEM((_CHUNK,), jnp.int32),
            pltpu.VMEM((_CHUNK, 16), jnp.float32),
            pltpu.VMEM((_CHUNK, 16), jnp.float32),
            pltpu.VMEM((_CHUNK, 16), jnp.float32),
            pltpu.VMEM((_CHUNK, 16), jnp.float32),
            pltpu.VMEM((_CHUNK, 16), jnp.float32),
            pltpu.SemaphoreType.DMA,
            pltpu.SemaphoreType.DMA,
        ],
    )(_sc_body)
    return run(table, edges).reshape(_E, 4, 4)


# speedup vs baseline: 4.5120x; 1.1762x over previous
import functools

import jax
import jax.numpy as jnp
from jax import lax
from jax.experimental import pallas as pl
from jax.experimental.pallas import tpu as pltpu
from jax.experimental.pallas import tpu_sc as plsc

_N_NODES = 100000
_E = 1600000
_NC = 2
_NS = 16
_NW = _NC * _NS
_CHUNK = 256
_EPW = _E // _NW
_NCHUNKS = -(-_EPW // _CHUNK)
_NCHUNKS += _NCHUNKS % 2


def _rsqrt(x):
    i = plsc.bitcast(x, jnp.int32)
    i = jnp.int32(0x5F3759DF) - (i >> 1)
    y = plsc.bitcast(i, jnp.float32)
    xh = x * 0.5
    for _ in range(3):
        y = y * (1.5 - xh * y * y)
    return y


def _cvec(v):
    return jnp.full((16,), v, jnp.int32)


def _compute_chunk(xi_b, xj_b, ob):
    def block(b, carry):
        rows = lax.iota(jnp.int32, 16) + b * 16
        qi = [[plsc.load_gather(xi_b, [rows, _cvec(4 * p + c)])
               for c in range(4)] for p in range(4)]
        qj = [[plsc.load_gather(xj_b, [rows, _cvec(4 * p + c)])
               for c in range(4)] for p in range(4)]
        for p in range(4):
            aw, ax, ay, az = qi[p]
            bw, bx, by, bz = qj[p]
            w = aw * bw + ax * bx + ay * by + az * bz
            x = ax * bw - aw * bx - ay * bz + az * by
            y = ay * bw - aw * by + ax * bz - az * bx
            z = az * bw - aw * bz - ax * by + ay * bx
            r = _rsqrt(w * w + x * x + y * y + z * z)
            for c, v in enumerate((w * r, x * r, y * r, z * r)):
                plsc.store_scatter(ob, [rows, _cvec(4 * p + c)], v)
        return carry

    lax.fori_loop(0, _CHUNK // 16, block, 0, unroll=4)


_ROWS_PER_SUB = _N_NODES // _NS


def _sc_body(table_hbm, edges_hbm, out_hbm, dump_hbm,
             table_sh, ii0, ij0, ii1, ij1, xi0, xj0, xi1, xj1, ob0, ob1,
             si0, si1, sg0, sg1, so0, so1):
    sid = lax.axis_index("s")
    wid = sid * _NC + lax.axis_index("c")
    wbase = wid * _EPW
    n = _NCHUNKS

    srow = sid * _ROWS_PER_SUB
    pltpu.sync_copy(table_hbm.at[pl.ds(srow, _ROWS_PER_SUB)],
                    table_sh.at[pl.ds(srow, _ROWS_PER_SUB)])
    plsc.subcore_barrier()

    def idx_issue(slot_ii, slot_ij, sem, base):
        pltpu.async_copy(edges_hbm.at[0, pl.ds(base, _CHUNK)], slot_ii, sem)
        pltpu.async_copy(edges_hbm.at[1, pl.ds(base, _CHUNK)], slot_ij, sem)

    def idx_wait(slot_ii, slot_ij, sem):
        pltpu.make_async_copy(
            edges_hbm.at[0, pl.ds(0, _CHUNK)], slot_ii, sem).wait()
        pltpu.make_async_copy(
            edges_hbm.at[1, pl.ds(0, _CHUNK)], slot_ij, sem).wait()

    def gather_issue(slot_ii, slot_ij, xi_b, xj_b, sem):
        pltpu.async_copy(table_sh.at[slot_ii], xi_b, sem)
        pltpu.async_copy(table_sh.at[slot_ij], xj_b, sem)

    def gather_wait(slot_ii, slot_ij, xi_b, xj_b, sem):
        pltpu.make_async_copy(table_sh.at[slot_ii], xi_b, sem).wait()
        pltpu.make_async_copy(table_sh.at[slot_ij], xj_b, sem).wait()

    def out_wait(ob, sem):
        pltpu.make_async_copy(ob, out_hbm.at[pl.ds(0, _CHUNK)], sem).wait()

    idx_issue(ii0, ij0, si0, wbase)
    idx_issue(ii1, ij1, si1, wbase + jnp.minimum(_CHUNK, _EPW - _CHUNK))
    idx_wait(ii0, ij0, si0)
    gather_issue(ii0, ij0, xi0, xj0, sg0)
    pltpu.async_copy(ob0, dump_hbm.at[wid, 0], so0)
    pltpu.async_copy(ob1, dump_hbm.at[wid, 1], so1)

    def half_iter(g, cur, nxt):
        (ii_c, ij_c, xi_c, xj_c, ob_c, si_c, sg_c, so_c) = cur
        (ii_n, ij_n, xi_n, xj_n, si_n, sg_n) = nxt
        base_g = wbase + jnp.minimum(g * _CHUNK, _EPW - _CHUNK)
        base_p = wbase + jnp.minimum((g + 2) * _CHUNK, _EPW - _CHUNK)
        idx_wait(ii_n, ij_n, si_n)
        gather_issue(ii_n, ij_n, xi_n, xj_n, sg_n)
        gather_wait(ii_c, ij_c, xi_c, xj_c, sg_c)
        idx_issue(ii_c, ij_c, si_c, base_p)
        out_wait(ob_c, so_c)
        _compute_chunk(xi_c, xj_c, ob_c)
        pltpu.async_copy(ob_c, out_hbm.at[pl.ds(base_g, _CHUNK)], so_c)

    slot0 = (ii0, ij0, xi0, xj0, ob0, si0, sg0, so0)
    slot1 = (ii1, ij1, xi1, xj1, ob1, si1, sg1, so1)
    nxt0 = (ii0, ij0, xi0, xj0, si0, sg0)
    nxt1 = (ii1, ij1, xi1, xj1, si1, sg1)

    def loop_body(t, carry):
        half_iter(2 * t, slot0, nxt1)
        half_iter(2 * t + 1, slot1, nxt0)
        return carry

    lax.fori_loop(0, n // 2, loop_body, 0)

    gather_wait(ii0, ij0, xi0, xj0, sg0)
    idx_wait(ii1, ij1, si1)
    out_wait(ob0, so0)
    out_wait(ob1, so1)


def kernel(particles, edges):
    table = particles.reshape(_N_NODES, 16)
    mesh = plsc.VectorSubcoreMesh(core_axis_name="c", subcore_axis_name="s")
    run = functools.partial(
        pl.kernel,
        mesh=mesh,
        compiler_params=pltpu.CompilerParams(
            use_tc_tiling_on_sc=False, needs_layout_passes=False),
        out_type=(jax.ShapeDtypeStruct((_E, 16), jnp.float32),
                  jax.ShapeDtypeStruct((_NW, 2, _CHUNK, 16), jnp.float32)),
        scratch_types=[
            pltpu.VMEM_SHARED((_N_NODES, 16), jnp.float32),
            pltpu.VMEM((_CHUNK,), jnp.int32),
            pltpu.VMEM((_CHUNK,), jnp.int32),
            pltpu.VMEM((_CHUNK,), jnp.int32),
            pltpu.VMEM((_CHUNK,), jnp.int32),
            pltpu.VMEM((_CHUNK, 16), jnp.float32),
            pltpu.VMEM((_CHUNK, 16), jnp.float32),
            pltpu.VMEM((_CHUNK, 16), jnp.float32),
            pltpu.VMEM((_CHUNK, 16), jnp.float32),
            pltpu.VMEM((_CHUNK, 16), jnp.float32),
            pltpu.VMEM((_CHUNK, 16), jnp.float32),
            pltpu.SemaphoreType.DMA,
            pltpu.SemaphoreType.DMA,
            pltpu.SemaphoreType.DMA,
            pltpu.SemaphoreType.DMA,
            pltpu.SemaphoreType.DMA,
            pltpu.SemaphoreType.DMA,
        ],
    )(_sc_body)
    out, _ = run(table, edges)
    return out.reshape(_E, 4, 4)


# speedup vs baseline: 4.5428x; 1.0068x over previous
import functools

import jax
import jax.numpy as jnp
from jax import lax
from jax.experimental import pallas as pl
from jax.experimental.pallas import tpu as pltpu
from jax.experimental.pallas import tpu_sc as plsc

_N_NODES = 100000
_E = 1600000
_NC = 2
_NS = 16
_NW = _NC * _NS
_CHUNK = 512
_EPW = _E // _NW
_NCHUNKS = -(-_EPW // _CHUNK)
_NCHUNKS += _NCHUNKS % 2


def _rsqrt(x):
    i = plsc.bitcast(x, jnp.int32)
    i = jnp.int32(0x5F3759DF) - (i >> 1)
    y = plsc.bitcast(i, jnp.float32)
    xh = x * 0.5
    for _ in range(3):
        y = y * (1.5 - xh * y * y)
    return y


def _cvec(v):
    return jnp.full((16,), v, jnp.int32)


def _compute_chunk(xi_b, xj_b, ob):
    def block(b, carry):
        rows = lax.iota(jnp.int32, 16) + b * 16
        qi = [[plsc.load_gather(xi_b, [rows, _cvec(4 * p + c)])
               for c in range(4)] for p in range(4)]
        qj = [[plsc.load_gather(xj_b, [rows, _cvec(4 * p + c)])
               for c in range(4)] for p in range(4)]
        for p in range(4):
            aw, ax, ay, az = qi[p]
            bw, bx, by, bz = qj[p]
            w = aw * bw + ax * bx + ay * by + az * bz
            x = ax * bw - aw * bx - ay * bz + az * by
            y = ay * bw - aw * by + ax * bz - az * bx
            z = az * bw - aw * bz - ax * by + ay * bx
            r = _rsqrt(w * w + x * x + y * y + z * z)
            for c, v in enumerate((w * r, x * r, y * r, z * r)):
                plsc.store_scatter(ob, [rows, _cvec(4 * p + c)], v)
        return carry

    lax.fori_loop(0, _CHUNK // 16, block, 0, unroll=4)


_ROWS_PER_SUB = _N_NODES // _NS


def _sc_body(table_hbm, edges_hbm, out_hbm, dump_hbm,
             ii0, ij0, ii1, ij1, xi0, xj0, xi1, xj1, ob0, ob1,
             si0, si1, sg0, sg1, so0, so1):
    wid = lax.axis_index("s") * _NC + lax.axis_index("c")
    wbase = wid * _EPW
    n = _NCHUNKS

    def idx_issue(slot_ii, slot_ij, sem, base):
        pltpu.async_copy(edges_hbm.at[0, pl.ds(base, _CHUNK)], slot_ii, sem)
        pltpu.async_copy(edges_hbm.at[1, pl.ds(base, _CHUNK)], slot_ij, sem)

    def idx_wait(slot_ii, slot_ij, sem):
        pltpu.make_async_copy(
            edges_hbm.at[0, pl.ds(0, _CHUNK)], slot_ii, sem).wait()
        pltpu.make_async_copy(
            edges_hbm.at[1, pl.ds(0, _CHUNK)], slot_ij, sem).wait()

    def gather_issue(slot_ii, slot_ij, xi_b, xj_b, sem):
        pltpu.async_copy(table_hbm.at[slot_ii], xi_b, sem)
        pltpu.async_copy(table_hbm.at[slot_ij], xj_b, sem)

    def gather_wait(slot_ii, slot_ij, xi_b, xj_b, sem):
        pltpu.make_async_copy(table_hbm.at[slot_ii], xi_b, sem).wait()
        pltpu.make_async_copy(table_hbm.at[slot_ij], xj_b, sem).wait()

    def out_wait(ob, sem):
        pltpu.make_async_copy(ob, out_hbm.at[pl.ds(0, _CHUNK)], sem).wait()

    idx_issue(ii0, ij0, si0, wbase)
    idx_issue(ii1, ij1, si1, wbase + jnp.minimum(_CHUNK, _EPW - _CHUNK))
    idx_wait(ii0, ij0, si0)
    gather_issue(ii0, ij0, xi0, xj0, sg0)
    pltpu.async_copy(ob0, dump_hbm.at[wid, 0], so0)
    pltpu.async_copy(ob1, dump_hbm.at[wid, 1], so1)

    def half_iter(g, cur, nxt):
        (ii_c, ij_c, xi_c, xj_c, ob_c, si_c, sg_c, so_c) = cur
        (ii_n, ij_n, xi_n, xj_n, si_n, sg_n) = nxt
        base_g = wbase + jnp.minimum(g * _CHUNK, _EPW - _CHUNK)
        base_p = wbase + jnp.minimum((g + 2) * _CHUNK, _EPW - _CHUNK)
        idx_wait(ii_n, ij_n, si_n)
        gather_issue(ii_n, ij_n, xi_n, xj_n, sg_n)
        gather_wait(ii_c, ij_c, xi_c, xj_c, sg_c)
        idx_issue(ii_c, ij_c, si_c, base_p)
        out_wait(ob_c, so_c)
        _compute_chunk(xi_c, xj_c, ob_c)
        pltpu.async_copy(ob_c, out_hbm.at[pl.ds(base_g, _CHUNK)], so_c)

    slot0 = (ii0, ij0, xi0, xj0, ob0, si0, sg0, so0)
    slot1 = (ii1, ij1, xi1, xj1, ob1, si1, sg1, so1)
    nxt0 = (ii0, ij0, xi0, xj0, si0, sg0)
    nxt1 = (ii1, ij1, xi1, xj1, si1, sg1)

    def loop_body(t, carry):
        half_iter(2 * t, slot0, nxt1)
        half_iter(2 * t + 1, slot1, nxt0)
        return carry

    lax.fori_loop(0, n // 2, loop_body, 0)

    gather_wait(ii0, ij0, xi0, xj0, sg0)
    idx_wait(ii1, ij1, si1)
    out_wait(ob0, so0)
    out_wait(ob1, so1)


def kernel(particles, edges):
    table = particles.reshape(_N_NODES, 16)
    mesh = plsc.VectorSubcoreMesh(core_axis_name="c", subcore_axis_name="s")
    run = functools.partial(
        pl.kernel,
        mesh=mesh,
        compiler_params=pltpu.CompilerParams(
            use_tc_tiling_on_sc=False, needs_layout_passes=False),
        out_type=(jax.ShapeDtypeStruct((_E, 16), jnp.float32),
                  jax.ShapeDtypeStruct((_NW, 2, _CHUNK, 16), jnp.float32)),
        scratch_types=[
            pltpu.VMEM((_CHUNK,), jnp.int32),
            pltpu.VMEM((_CHUNK,), jnp.int32),
            pltpu.VMEM((_CHUNK,), jnp.int32),
            pltpu.VMEM((_CHUNK,), jnp.int32),
            pltpu.VMEM((_CHUNK, 16), jnp.float32),
            pltpu.VMEM((_CHUNK, 16), jnp.float32),
            pltpu.VMEM((_CHUNK, 16), jnp.float32),
            pltpu.VMEM((_CHUNK, 16), jnp.float32),
            pltpu.VMEM((_CHUNK, 16), jnp.float32),
            pltpu.VMEM((_CHUNK, 16), jnp.float32),
            pltpu.SemaphoreType.DMA,
            pltpu.SemaphoreType.DMA,
            pltpu.SemaphoreType.DMA,
            pltpu.SemaphoreType.DMA,
            pltpu.SemaphoreType.DMA,
            pltpu.SemaphoreType.DMA,
        ],
    )(_sc_body)
    out, _ = run(table, edges)
    return out.reshape(_E, 4, 4)
